# baseline (speedup 1.0000x reference)
import jax
import jax.numpy as jnp
from jax import lax
from jax.experimental import pallas as pl
from jax.experimental.pallas import tpu as pltpu
from jax.experimental.pallas import tpu_sc as plsc

VOCAB = 100000
EMB = 64
CLS = 4
BATCH = 4096
HIST = 200

LANES = 16
PACK = 8
KDIM = EMB * PACK
NDIM = LANES * PACK
MROWS = VOCAB // PACK
BM = 1024

NWORKERS = 32
ITEMS = BATCH // NWORKERS
NSLOTS = 8
NGROUPS = HIST // NSLOTS
ROWPAD = 208


BN = 8192


def _project_body(w_ref, t_ref, o_ref):
    o_ref[...] = jax.lax.dot_general(
        w_ref[...],
        t_ref[...],
        (((1,), (0,)), ((), ())),
        preferred_element_type=jnp.float32,
    )


def _project(w16, t_t):
    grid = pl.cdiv(VOCAB, BN)
    return pl.pallas_call(
        _project_body,
        grid=(grid,),
        in_specs=[
            pl.BlockSpec((LANES, EMB), lambda i: (0, 0)),
            pl.BlockSpec((EMB, BN), lambda i: (0, i)),
        ],
        out_specs=pl.BlockSpec((LANES, BN), lambda i: (0, i)),
        out_shape=jax.ShapeDtypeStruct((LANES, VOCAB), jnp.float32),
    )(w16, t_t)


def _sc_body(p16_hbm, data_hbm, out_hbm, rowidx_v, idx_pad_v, accs, sems):
    wid = lax.axis_index("s") * 2 + lax.axis_index("c")
    base = wid * ITEMS

    nrow = ROWPAD
    for k in range(nrow // 16):
        r = jnp.minimum(
            (lax.iota(jnp.int32, 16) + 16 * k) * NWORKERS + wid,
            BATCH * HIST // 128 - 1,
        )
        rowidx_v[pl.ds(16 * k, 16)] = r
    pltpu.async_copy(data_hbm.at[rowidx_v], idx_pad_v, sems[0]).wait()
    idx_v = idx_pad_v

    for g in range(NSLOTS):
        pltpu.async_copy(p16_hbm.at[idx_v.at[g]], accs[g], sems[g])

    @pl.loop(1, NGROUPS)
    def _(i):
        for g in range(NSLOTS):
            l = i * NSLOTS + g
            pltpu.make_async_copy(p16_hbm.at[idx_v.at[g]], accs[g], sems[g]).wait()
            pltpu.async_copy(p16_hbm.at[idx_v.at[l]], accs[g], sems[g], add=True)

    for g in range(NSLOTS):
        pltpu.make_async_copy(p16_hbm.at[idx_v.at[g]], accs[g], sems[g]).wait()

    @pl.loop(0, ITEMS)
    def _(j):
        tot = accs[0][j]
        for g in range(1, NSLOTS):
            tot = tot + accs[g][j]
        accs[0][j] = tot

    pltpu.sync_copy(accs[0], out_hbm.at[pl.ds(base, ITEMS)])


def _pooled_lookup(p16, d128):
    mesh = plsc.VectorSubcoreMesh(core_axis_name="c", subcore_axis_name="s")
    scratch = (
        pltpu.VMEM((ROWPAD,), jnp.int32),
        pltpu.VMEM((ROWPAD, ITEMS), jnp.int32),
        tuple(pltpu.VMEM((ITEMS, LANES), jnp.float32) for _ in range(NSLOTS)),
        tuple(pltpu.SemaphoreType.DMA for _ in range(NSLOTS)),
    )
    f = pl.kernel(
        _sc_body,
        out_type=jax.ShapeDtypeStruct((BATCH, LANES), jnp.float32),
        mesh=mesh,
        scratch_types=scratch,
        compiler_params=pltpu.CompilerParams(
            use_tc_tiling_on_sc=False, needs_layout_passes=False
        ),
    )
    return f(p16, d128)


@jax.jit
def kernel(data, table, W, b):
    w16 = jnp.zeros((LANES, EMB), jnp.float32).at[:CLS].set(W / HIST)

    p16 = _project(w16, table.T).T
    d128 = data.T.reshape(BATCH * HIST // 128, 128)
    out16 = _pooled_lookup(p16, d128)
    return out16[:, :CLS] + b


# speedup vs baseline: 1.3438x; 1.3438x over previous
import jax
import jax.numpy as jnp
from jax import lax
from jax.experimental import pallas as pl
from jax.experimental.pallas import tpu as pltpu
from jax.experimental.pallas import tpu_sc as plsc

VOCAB = 100000
EMB = 64
CLS = 4
BATCH = 4096
HIST = 200

LANES = 16
NWORKERS = 32
ITEMS = BATCH // NWORKERS

VP = 102400
VCHUNK = VP // NWORKERS
BN = 5120

NSLOTS = 8
NGROUPS = HIST // NSLOTS
ROWPAD = 208


def _project_body(w_ref, t_ref, o0, o1, o2, o3):
    o = jax.lax.dot_general(
        w_ref[...],
        t_ref[...],
        (((1,), (0,)), ((), ())),
        preferred_element_type=jnp.float32,
    )
    for c, o_ref in enumerate((o0, o1, o2, o3)):
        o_ref[...] = o[c, :]


def _project(w8, t_t):
    grid = VP // BN
    out = jax.ShapeDtypeStruct((VP,), jnp.float32)
    return pl.pallas_call(
        _project_body,
        grid=(grid,),
        in_specs=[
            pl.BlockSpec((8, EMB), lambda i: (0, 0)),
            pl.BlockSpec((EMB, BN), lambda i: (0, i)),
        ],
        out_specs=[pl.BlockSpec((BN,), lambda i: (i,)) for _ in range(CLS)],
        out_shape=[out, out, out, out],
    )(w8, t_t)


def _interleave_body(p0, p1, p2, p3, out_hbm, cols, acc, sem):
    wid = lax.axis_index("s") * 2 + lax.axis_index("c")
    vb = wid * VCHUNK
    for c, src in enumerate((p0, p1, p2, p3)):
        pltpu.async_copy(src.at[pl.ds(vb, VCHUNK)], cols.at[c], sem)
    for c in range(CLS):
        pltpu.make_async_copy(p0.at[pl.ds(vb, VCHUNK)], cols.at[c], sem).wait()

    zeros = jnp.zeros((16,), jnp.float32)
    for c in range(LANES):
        ccol = jnp.full((16,), c, jnp.int32)

        @pl.loop(0, VCHUNK // LANES)
        def _(k):
            rows = lax.iota(jnp.int32, 16) + k * LANES
            vals = cols[c, pl.ds(k * LANES, LANES)] if c < CLS else zeros
            plsc.store_scatter(acc, [rows, ccol], vals)

    pltpu.sync_copy(acc, out_hbm.at[pl.ds(vb, VCHUNK)])


def _interleave(p0, p1, p2, p3):
    mesh = plsc.VectorSubcoreMesh(core_axis_name="c", subcore_axis_name="s")
    scratch = (
        pltpu.VMEM((CLS, VCHUNK), jnp.float32),
        pltpu.VMEM((VCHUNK, LANES), jnp.float32),
        pltpu.SemaphoreType.DMA,
    )
    f = pl.kernel(
        _interleave_body,
        out_type=jax.ShapeDtypeStruct((VP, LANES), jnp.float32),
        mesh=mesh,
        scratch_types=scratch,
        compiler_params=pltpu.CompilerParams(
            use_tc_tiling_on_sc=False, needs_layout_passes=False
        ),
    )
    return f(p0, p1, p2, p3)


def _sc_body(p16_hbm, data_hbm, out_hbm, rowidx_v, idx_v, accs, sems):
    wid = lax.axis_index("s") * 2 + lax.axis_index("c")
    base = wid * ITEMS

    for k in range(ROWPAD // 16):
        r = jnp.minimum(
            (lax.iota(jnp.int32, 16) + 16 * k) * NWORKERS + wid,
            BATCH * HIST // 128 - 1,
        )
        rowidx_v[pl.ds(16 * k, 16)] = r
    pltpu.async_copy(data_hbm.at[rowidx_v], idx_v, sems[0]).wait()

    for g in range(NSLOTS):
        pltpu.async_copy(p16_hbm.at[idx_v.at[g]], accs[g], sems[g])

    @pl.loop(1, NGROUPS)
    def _(i):
        for g in range(NSLOTS):
            l = i * NSLOTS + g
            pltpu.make_async_copy(p16_hbm.at[idx_v.at[g]], accs[g], sems[g]).wait()
            pltpu.async_copy(p16_hbm.at[idx_v.at[l]], accs[g], sems[g], add=True)

    for g in range(NSLOTS):
        pltpu.make_async_copy(p16_hbm.at[idx_v.at[g]], accs[g], sems[g]).wait()

    @pl.loop(0, ITEMS)
    def _(j):
        tot = accs[0][j]
        for g in range(1, NSLOTS):
            tot = tot + accs[g][j]
        accs[0][j] = tot

    pltpu.sync_copy(accs[0], out_hbm.at[pl.ds(base, ITEMS)])


def _pooled_lookup(p4, d128):
    mesh = plsc.VectorSubcoreMesh(core_axis_name="c", subcore_axis_name="s")
    scratch = (
        pltpu.VMEM((ROWPAD,), jnp.int32),
        pltpu.VMEM((ROWPAD, ITEMS), jnp.int32),
        tuple(pltpu.VMEM((ITEMS, LANES), jnp.float32) for _ in range(NSLOTS)),
        tuple(pltpu.SemaphoreType.DMA for _ in range(NSLOTS)),
    )
    f = pl.kernel(
        _sc_body,
        out_type=jax.ShapeDtypeStruct((BATCH, LANES), jnp.float32),
        mesh=mesh,
        scratch_types=scratch,
        compiler_params=pltpu.CompilerParams(
            use_tc_tiling_on_sc=False, needs_layout_passes=False
        ),
    )
    return f(p4, d128)


@jax.jit
def kernel(data, table, W, b):
    w8 = jnp.zeros((8, EMB), jnp.float32).at[:CLS].set(W / HIST)

    p0, p1, p2, p3 = _project(w8, table.T)
    p16 = _interleave(p0, p1, p2, p3)

    d128 = data.T.reshape(BATCH * HIST // 128, 128)
    out16 = _pooled_lookup(p16, d128)
    return out16[:, :CLS] + b


# speedup vs baseline: 1.4694x; 1.0935x over previous
import jax
import jax.numpy as jnp
from jax import lax
from jax.experimental import pallas as pl
from jax.experimental.pallas import tpu as pltpu
from jax.experimental.pallas import tpu_sc as plsc

VOCAB = 100000
EMB = 64
CLS = 4
BATCH = 4096
HIST = 200

LANES = 16
NWORKERS = 32
ITEMS = BATCH // NWORKERS

VP = 102400
VCHUNK = VP // NWORKERS
BN = 5120

NSLOTS = 20
NGROUPS = HIST // NSLOTS
ROWPAD = 208


def _project_body(w_ref, t_ref, o0, o1, o2, o3):
    o = jax.lax.dot_general(
        w_ref[...],
        t_ref[...],
        (((1,), (0,)), ((), ())),
        preferred_element_type=jnp.float32,
    )
    for c, o_ref in enumerate((o0, o1, o2, o3)):
        o_ref[...] = o[c, :]


def _project(w8, t_t):
    grid = VP // BN
    out = jax.ShapeDtypeStruct((VP,), jnp.float32)
    return pl.pallas_call(
        _project_body,
        grid=(grid,),
        in_specs=[
            pl.BlockSpec((8, EMB), lambda i: (0, 0)),
            pl.BlockSpec((EMB, BN), lambda i: (0, i)),
        ],
        out_specs=[pl.BlockSpec((BN,), lambda i: (i,)) for _ in range(CLS)],
        out_shape=[out, out, out, out],
    )(w8, t_t)


def _interleave_body(p0, p1, p2, p3, out_hbm, cols, acc, sem):
    wid = lax.axis_index("s") * 2 + lax.axis_index("c")
    vb = wid * VCHUNK
    for c, src in enumerate((p0, p1, p2, p3)):
        pltpu.async_copy(src.at[pl.ds(vb, VCHUNK)], cols.at[c], sem)
    for c in range(CLS):
        pltpu.make_async_copy(p0.at[pl.ds(vb, VCHUNK)], cols.at[c], sem).wait()

    for c in range(CLS):
        ccol = jnp.full((16,), c, jnp.int32)

        @pl.loop(0, VCHUNK // LANES, unroll=4)
        def _(k):
            rows = lax.iota(jnp.int32, 16) + k * LANES
            vals = cols[c, pl.ds(k * LANES, LANES)]
            plsc.store_scatter(acc, [rows, ccol], vals)

    pltpu.sync_copy(acc, out_hbm.at[pl.ds(vb, VCHUNK)])


def _interleave(p0, p1, p2, p3):
    mesh = plsc.VectorSubcoreMesh(core_axis_name="c", subcore_axis_name="s")
    scratch = (
        pltpu.VMEM((CLS, VCHUNK), jnp.float32),
        pltpu.VMEM((VCHUNK, LANES), jnp.float32),
        pltpu.SemaphoreType.DMA,
    )
    f = pl.kernel(
        _interleave_body,
        out_type=jax.ShapeDtypeStruct((VP, LANES), jnp.float32),
        mesh=mesh,
        scratch_types=scratch,
        compiler_params=pltpu.CompilerParams(
            use_tc_tiling_on_sc=False, needs_layout_passes=False
        ),
    )
    return f(p0, p1, p2, p3)


def _sc_body(p16_hbm, data_hbm, out_hbm, rowidx_v, idx_v, accs, sems):
    wid = lax.axis_index("s") * 2 + lax.axis_index("c")
    base = wid * ITEMS

    for k in range(ROWPAD // 16):
        r = jnp.minimum(
            (lax.iota(jnp.int32, 16) + 16 * k) * NWORKERS + wid,
            BATCH * HIST // 128 - 1,
        )
        rowidx_v[pl.ds(16 * k, 16)] = r
    pltpu.async_copy(data_hbm.at[rowidx_v], idx_v, sems[0]).wait()

    for g in range(NSLOTS):
        pltpu.async_copy(p16_hbm.at[idx_v.at[g]], accs[g], sems[g])

    @pl.loop(1, NGROUPS)
    def _(i):
        for g in range(NSLOTS):
            l = i * NSLOTS + g
            pltpu.make_async_copy(p16_hbm.at[idx_v.at[g]], accs[g], sems[g]).wait()
            pltpu.async_copy(p16_hbm.at[idx_v.at[l]], accs[g], sems[g], add=True)

    for g in range(NSLOTS):
        pltpu.make_async_copy(p16_hbm.at[idx_v.at[g]], accs[g], sems[g]).wait()

    @pl.loop(0, ITEMS)
    def _(j):
        tot = accs[0][j]
        for g in range(1, NSLOTS):
            tot = tot + accs[g][j]
        accs[0][j] = tot

    pltpu.sync_copy(accs[0], out_hbm.at[pl.ds(base, ITEMS)])


def _pooled_lookup(p4, d128):
    mesh = plsc.VectorSubcoreMesh(core_axis_name="c", subcore_axis_name="s")
    scratch = (
        pltpu.VMEM((ROWPAD,), jnp.int32),
        pltpu.VMEM((ROWPAD, ITEMS), jnp.int32),
        tuple(pltpu.VMEM((ITEMS, LANES), jnp.float32) for _ in range(NSLOTS)),
        tuple(pltpu.SemaphoreType.DMA for _ in range(NSLOTS)),
    )
    f = pl.kernel(
        _sc_body,
        out_type=jax.ShapeDtypeStruct((BATCH, LANES), jnp.float32),
        mesh=mesh,
        scratch_types=scratch,
        compiler_params=pltpu.CompilerParams(
            use_tc_tiling_on_sc=False, needs_layout_passes=False
        ),
    )
    return f(p4, d128)


@jax.jit
def kernel(data, table, W, b):
    w8 = jnp.zeros((8, EMB), jnp.float32).at[:CLS].set(W / HIST)

    p0, p1, p2, p3 = _project(w8, table.T)
    p16 = _interleave(p0, p1, p2, p3)

    d128 = data.T.reshape(BATCH * HIST // 128, 128)
    out16 = _pooled_lookup(p16, d128)
    return out16[:, :CLS] + b


# speedup vs baseline: 1.9338x; 1.3160x over previous
import jax
import jax.numpy as jnp
from jax import lax
from jax.experimental import pallas as pl
from jax.experimental.pallas import tpu as pltpu
from jax.experimental.pallas import tpu_sc as plsc

VOCAB = 100000
EMB = 64
CLS = 4
BATCH = 4096
HIST = 200

LANES = 16
NWORKERS = 32
ITEMS = BATCH // NWORKERS

VP = 102400
VCHUNK = VP // NWORKERS
BN = 5120

NSLOTS = 8
NGROUPS = HIST // NSLOTS
ROWPAD = 208
ROWW = 8


def _project_body(w_ref, t_ref, o0, o1, o2, o3):
    o = jax.lax.dot_general(
        w_ref[...],
        t_ref[...],
        (((1,), (0,)), ((), ())),
        preferred_element_type=jnp.float32,
    )
    for c, o_ref in enumerate((o0, o1, o2, o3)):
        o_ref[...] = o[c, :]


def _project(w8, t_t):
    grid = VP // BN
    out = jax.ShapeDtypeStruct((VP,), jnp.float32)
    return pl.pallas_call(
        _project_body,
        grid=(grid,),
        in_specs=[
            pl.BlockSpec((8, EMB), lambda i: (0, 0)),
            pl.BlockSpec((EMB, BN), lambda i: (0, i)),
        ],
        out_specs=[pl.BlockSpec((BN,), lambda i: (i,)) for _ in range(CLS)],
        out_shape=[out, out, out, out],
    )(w8, t_t)


def _interleave_body(p0, p1, p2, p3, out_hbm, cols, acc, sem):
    wid = lax.axis_index("s") * 2 + lax.axis_index("c")
    vb = wid * VCHUNK
    for c, src in enumerate((p0, p1, p2, p3)):
        pltpu.async_copy(src.at[pl.ds(vb, VCHUNK)], cols.at[c], sem)
    for c in range(CLS):
        pltpu.make_async_copy(p0.at[pl.ds(vb, VCHUNK)], cols.at[c], sem).wait()

    for c in range(CLS):
        ccol = jnp.full((16,), c, jnp.int32)

        @pl.loop(0, VCHUNK // LANES, unroll=4)
        def _(k):
            rows = lax.iota(jnp.int32, 16) + k * LANES
            vals = cols[c, pl.ds(k * LANES, LANES)]
            plsc.store_scatter(acc, [rows, ccol], vals)

    pltpu.sync_copy(acc, out_hbm.at[pl.ds(vb, VCHUNK)])


def _interleave(p0, p1, p2, p3):
    mesh = plsc.VectorSubcoreMesh(core_axis_name="c", subcore_axis_name="s")
    scratch = (
        pltpu.VMEM((CLS, VCHUNK), jnp.float32),
        pltpu.VMEM((VCHUNK, ROWW), jnp.float32),
        pltpu.SemaphoreType.DMA,
    )
    f = pl.kernel(
        _interleave_body,
        out_type=jax.ShapeDtypeStruct((VP, ROWW), jnp.float32),
        mesh=mesh,
        scratch_types=scratch,
        compiler_params=pltpu.CompilerParams(
            use_tc_tiling_on_sc=False, needs_layout_passes=False
        ),
    )
    return f(p0, p1, p2, p3)


def _sc_body(p16_hbm, data_hbm, out_hbm, rowidx_v, idx_v, accs, shared, sems):
    wid = lax.axis_index("s") * 2 + lax.axis_index("c")
    sid = lax.axis_index("s")
    base = wid * ITEMS

    vchunk16 = VP // 16
    pltpu.sync_copy(
        p16_hbm.at[pl.ds(sid * vchunk16, vchunk16)],
        shared.at[pl.ds(sid * vchunk16, vchunk16)],
    )

    for k in range(ROWPAD // 16):
        r = jnp.minimum(
            (lax.iota(jnp.int32, 16) + 16 * k) * NWORKERS + wid,
            BATCH * HIST // 128 - 1,
        )
        rowidx_v[pl.ds(16 * k, 16)] = r
    pltpu.async_copy(data_hbm.at[rowidx_v], idx_v, sems[0]).wait()

    plsc.subcore_barrier()

    for g in range(NSLOTS):
        pltpu.async_copy(shared.at[idx_v.at[g]], accs[g], sems[g])

    @pl.loop(1, NGROUPS)
    def _(i):
        for g in range(NSLOTS):
            l = i * NSLOTS + g
            pltpu.make_async_copy(shared.at[idx_v.at[g]], accs[g], sems[g]).wait()
            pltpu.async_copy(shared.at[idx_v.at[l]], accs[g], sems[g], add=True)

    for g in range(NSLOTS):
        pltpu.make_async_copy(shared.at[idx_v.at[g]], accs[g], sems[g]).wait()

    @pl.loop(0, ITEMS * ROWW // 16)
    def _(k):
        q = lax.iota(jnp.int32, 16) + k * 16
        rows = lax.shift_right_logical(q, 3)
        ccs = lax.bitwise_and(q, 7)
        tot = plsc.load_gather(accs[0], [rows, ccs])
        for g in range(1, NSLOTS):
            tot = tot + plsc.load_gather(accs[g], [rows, ccs])
        plsc.store_scatter(accs[0], [rows, ccs], tot)

    pltpu.sync_copy(accs[0], out_hbm.at[pl.ds(base, ITEMS)])


def _pooled_lookup(p4, d128):
    mesh = plsc.VectorSubcoreMesh(core_axis_name="c", subcore_axis_name="s")
    scratch = (
        pltpu.VMEM((ROWPAD,), jnp.int32),
        pltpu.VMEM((ROWPAD, ITEMS), jnp.int32),
        tuple(pltpu.VMEM((ITEMS, ROWW), jnp.float32) for _ in range(NSLOTS)),
        pltpu.VMEM_SHARED((VP, ROWW), jnp.float32),
        tuple(pltpu.SemaphoreType.DMA for _ in range(NSLOTS)),
    )
    f = pl.kernel(
        _sc_body,
        out_type=jax.ShapeDtypeStruct((BATCH, ROWW), jnp.float32),
        mesh=mesh,
        scratch_types=scratch,
        compiler_params=pltpu.CompilerParams(
            use_tc_tiling_on_sc=False, needs_layout_passes=False
        ),
    )
    return f(p4, d128)


@jax.jit
def kernel(data, table, W, b):
    w8 = jnp.zeros((8, EMB), jnp.float32).at[:CLS].set(W / HIST)

    p0, p1, p2, p3 = _project(w8, table.T)
    p16 = _interleave(p0, p1, p2, p3)

    d128 = data.T.reshape(BATCH * HIST // 128, 128)
    out16 = _pooled_lookup(p16, d128)
    return out16[:, :CLS] + b


# speedup vs baseline: 1.9891x; 1.0286x over previous
import jax
import jax.numpy as jnp
from jax import lax
from jax.experimental import pallas as pl
from jax.experimental.pallas import tpu as pltpu
from jax.experimental.pallas import tpu_sc as plsc

VOCAB = 100000
EMB = 64
CLS = 4
BATCH = 4096
HIST = 200

LANES = 16
NWORKERS = 32
ITEMS = BATCH // NWORKERS

VP = 102400
N16 = VP // 16
ICHUNK = 3200
BN = 5120

NSLOTS = 8
NGROUPS = HIST // NSLOTS
ROWPAD = 208
ROWW = 8


def _project_body(w_ref, t_ref, o0, o1, o2, o3):
    o = jax.lax.dot_general(
        w_ref[...],
        t_ref[...],
        (((1,), (0,)), ((), ())),
        preferred_element_type=jnp.float32,
    )
    for c, o_ref in enumerate((o0, o1, o2, o3)):
        o_ref[...] = o[c, :]


def _project(w8, t_t):
    grid = VP // BN
    out = jax.ShapeDtypeStruct((VP,), jnp.float32)
    return pl.pallas_call(
        _project_body,
        grid=(grid,),
        in_specs=[
            pl.BlockSpec((8, EMB), lambda i: (0, 0)),
            pl.BlockSpec((EMB, BN), lambda i: (0, i)),
        ],
        out_specs=[pl.BlockSpec((BN,), lambda i: (i,)) for _ in range(CLS)],
        out_shape=[out, out, out, out],
    )(w8, t_t)


def _sc_body(p0, p1, p2, p3, data_hbm, out_hbm,
             rowidx_v, idx_v, cols_v, ichunk_v, accs, shared, sems, ilsem):
    wid = lax.axis_index("s") * 2 + lax.axis_index("c")
    sid = lax.axis_index("s")
    base = wid * ITEMS

    for k in range(ROWPAD // 16):
        r = jnp.minimum(
            (lax.iota(jnp.int32, 16) + 16 * k) * NWORKERS + wid,
            BATCH * HIST // 128 - 1,
        )
        rowidx_v[pl.ds(16 * k, 16)] = r
    pltpu.async_copy(data_hbm.at[rowidx_v], idx_v, sems[0])

    for h in range(N16 // ICHUNK):
        vb = sid * N16 + h * ICHUNK
        for c, src in enumerate((p0, p1, p2, p3)):
            pltpu.async_copy(src.at[pl.ds(vb, ICHUNK)], cols_v.at[c], ilsem)
        for c in range(CLS):
            pltpu.make_async_copy(p0.at[pl.ds(vb, ICHUNK)], cols_v.at[c], ilsem).wait()
        for c in range(CLS):
            ccol = jnp.full((16,), c, jnp.int32)

            @pl.loop(0, ICHUNK // LANES, unroll=4)
            def _(k):
                rows = lax.iota(jnp.int32, 16) + k * LANES
                vals = cols_v[c, pl.ds(k * LANES, LANES)]
                plsc.store_scatter(ichunk_v, [rows, ccol], vals)

        pltpu.sync_copy(ichunk_v, shared.at[pl.ds(vb, ICHUNK)])

    plsc.subcore_barrier()
    pltpu.make_async_copy(data_hbm.at[rowidx_v], idx_v, sems[0]).wait()

    for g in range(NSLOTS):
        pltpu.async_copy(shared.at[idx_v.at[g]], accs[g], sems[g])

    @pl.loop(1, NGROUPS)
    def _(i):
        for g in range(NSLOTS):
            l = i * NSLOTS + g
            pltpu.make_async_copy(shared.at[idx_v.at[g]], accs[g], sems[g]).wait()
            pltpu.async_copy(shared.at[idx_v.at[l]], accs[g], sems[g], add=True)

    for g in range(NSLOTS):
        pltpu.make_async_copy(shared.at[idx_v.at[g]], accs[g], sems[g]).wait()

    @pl.loop(0, ITEMS * ROWW // 16)
    def _(k):
        q = lax.iota(jnp.int32, 16) + k * 16
        rows = lax.shift_right_logical(q, 3)
        ccs = lax.bitwise_and(q, 7)
        tot = plsc.load_gather(accs[0], [rows, ccs])
        for g in range(1, NSLOTS):
            tot = tot + plsc.load_gather(accs[g], [rows, ccs])
        plsc.store_scatter(accs[0], [rows, ccs], tot)

    pltpu.sync_copy(accs[0], out_hbm.at[pl.ds(base, ITEMS)])


def _pooled_lookup(p0, p1, p2, p3, d128):
    mesh = plsc.VectorSubcoreMesh(core_axis_name="c", subcore_axis_name="s")
    scratch = (
        pltpu.VMEM((ROWPAD,), jnp.int32),
        pltpu.VMEM((ROWPAD, ITEMS), jnp.int32),
        pltpu.VMEM((CLS, ICHUNK), jnp.float32),
        pltpu.VMEM((ICHUNK, ROWW), jnp.float32),
        tuple(pltpu.VMEM((ITEMS, ROWW), jnp.float32) for _ in range(NSLOTS)),
        pltpu.VMEM_SHARED((VP, ROWW), jnp.float32),
        tuple(pltpu.SemaphoreType.DMA for _ in range(NSLOTS)),
        pltpu.SemaphoreType.DMA,
    )
    f = pl.kernel(
        _sc_body,
        out_type=jax.ShapeDtypeStruct((BATCH, ROWW), jnp.float32),
        mesh=mesh,
        scratch_types=scratch,
        compiler_params=pltpu.CompilerParams(
            use_tc_tiling_on_sc=False, needs_layout_passes=False
        ),
    )
    return f(p0, p1, p2, p3, d128)


@jax.jit
def kernel(data, table, W, b):
    w8 = jnp.zeros((8, EMB), jnp.float32).at[:CLS].set(W / HIST)

    p0, p1, p2, p3 = _project(w8, table.T)

    d128 = data.T.reshape(BATCH * HIST // 128, 128)
    out8 = _pooled_lookup(p0, p1, p2, p3, d128)
    return out8[:, :CLS] + b


# speedup vs baseline: 2.1623x; 1.0871x over previous
import jax
import jax.numpy as jnp
from jax import lax
from jax.experimental import pallas as pl
from jax.experimental.pallas import tpu as pltpu
from jax.experimental.pallas import tpu_sc as plsc

VOCAB = 100000
EMB = 64
CLS = 4
BATCH = 4096
HIST = 200

LANES = 16
NWORKERS = 32
ITEMS = BATCH // NWORKERS

VP = 102400
N16 = VP // 16
ICHUNK = 3200
BN = 10240

NSLOTS = 8
NGROUPS = HIST // NSLOTS
ROWPAD = 208
ROWW = 8


def _project_body(w_ref, t_ref, o0, o1, o2, o3):
    o = jax.lax.dot_general(
        w_ref[...],
        t_ref[...],
        (((1,), (0,)), ((), ())),
        preferred_element_type=jnp.float32,
    )
    for c, o_ref in enumerate((o0, o1, o2, o3)):
        o_ref[...] = o[c, :]


def _project(w8, t_t):
    grid = VP // BN
    out = jax.ShapeDtypeStruct((VP,), jnp.float32)
    return pl.pallas_call(
        _project_body,
        grid=(grid,),
        in_specs=[
            pl.BlockSpec((8, EMB), lambda i: (0, 0)),
            pl.BlockSpec((EMB, BN), lambda i: (0, i)),
        ],
        out_specs=[pl.BlockSpec((BN,), lambda i: (i,)) for _ in range(CLS)],
        out_shape=[out, out, out, out],
    )(w8, t_t)


def _sc_body(p0, p1, p2, p3, data_hbm, out_hbm,
             rowidx_v, idx_v, cols_v, ichunk_v, accs, shared, sems, ilsem):
    wid = lax.axis_index("s") * 2 + lax.axis_index("c")
    sid = lax.axis_index("s")
    base = wid * ITEMS

    for k in range(ROWPAD // 16):
        r = jnp.minimum(
            (lax.iota(jnp.int32, 16) + 16 * k) * NWORKERS + wid,
            BATCH * HIST // 128 - 1,
        )
        rowidx_v[pl.ds(16 * k, 16)] = r
    pltpu.async_copy(data_hbm.at[rowidx_v], idx_v, sems[0])

    for h in range(N16 // ICHUNK):
        vb = sid * N16 + h * ICHUNK
        for c, src in enumerate((p0, p1, p2, p3)):
            pltpu.async_copy(src.at[pl.ds(vb, ICHUNK)], cols_v.at[c], ilsem)
        for c in range(CLS):
            pltpu.make_async_copy(p0.at[pl.ds(vb, ICHUNK)], cols_v.at[c], ilsem).wait()
        for c in range(CLS):
            ccol = jnp.full((16,), c, jnp.int32)

            @pl.loop(0, ICHUNK // LANES, unroll=4)
            def _(k):
                rows = lax.iota(jnp.int32, 16) + k * LANES
                vals = cols_v[c, pl.ds(k * LANES, LANES)]
                plsc.store_scatter(ichunk_v, [rows, ccol], vals)

        pltpu.sync_copy(ichunk_v, shared.at[pl.ds(vb, ICHUNK)])

    plsc.subcore_barrier()
    pltpu.make_async_copy(data_hbm.at[rowidx_v], idx_v, sems[0]).wait()

    for g in range(NSLOTS):
        pltpu.async_copy(shared.at[idx_v.at[g]], accs[g], sems[g])

    @pl.loop(1, NGROUPS)
    def _(i):
        for g in range(NSLOTS):
            l = i * NSLOTS + g
            pltpu.make_async_copy(shared.at[idx_v.at[g]], accs[g], sems[g]).wait()
            pltpu.async_copy(shared.at[idx_v.at[l]], accs[g], sems[g], add=True)

    for g in range(NSLOTS):
        pltpu.make_async_copy(shared.at[idx_v.at[g]], accs[g], sems[g]).wait()

    @pl.loop(0, ITEMS * ROWW // 16)
    def _(k):
        q = lax.iota(jnp.int32, 16) + k * 16
        rows = lax.shift_right_logical(q, 3)
        ccs = lax.bitwise_and(q, 7)
        tot = plsc.load_gather(accs[0], [rows, ccs])
        for g in range(1, NSLOTS):
            tot = tot + plsc.load_gather(accs[g], [rows, ccs])
        plsc.store_scatter(accs[0], [rows, ccs], tot)

    pltpu.sync_copy(accs[0], out_hbm.at[pl.ds(base, ITEMS)])


def _pooled_lookup(p0, p1, p2, p3, d128):
    mesh = plsc.VectorSubcoreMesh(core_axis_name="c", subcore_axis_name="s")
    scratch = (
        pltpu.VMEM((ROWPAD,), jnp.int32),
        pltpu.VMEM((ROWPAD, ITEMS), jnp.int32),
        pltpu.VMEM((CLS, ICHUNK), jnp.float32),
        pltpu.VMEM((ICHUNK, ROWW), jnp.float32),
        tuple(pltpu.VMEM((ITEMS, ROWW), jnp.float32) for _ in range(NSLOTS)),
        pltpu.VMEM_SHARED((VP, ROWW), jnp.float32),
        tuple(pltpu.SemaphoreType.DMA for _ in range(NSLOTS)),
        pltpu.SemaphoreType.DMA,
    )
    f = pl.kernel(
        _sc_body,
        out_type=jax.ShapeDtypeStruct((BATCH, ROWW), jnp.float32),
        mesh=mesh,
        scratch_types=scratch,
        compiler_params=pltpu.CompilerParams(
            use_tc_tiling_on_sc=False, needs_layout_passes=False
        ),
    )
    return f(p0, p1, p2, p3, d128)


@jax.jit
def kernel(data, table, W, b):
    w8 = jnp.zeros((8, EMB), jnp.float32).at[:CLS].set(W / HIST)

    p0, p1, p2, p3 = _project(w8, table.T)

    d128 = data.T.reshape(BATCH * HIST // 128, 128)
    out8 = _pooled_lookup(p0, p1, p2, p3, d128)
    return out8[:, :CLS] + b


# speedup vs baseline: 2.2320x; 1.0322x over previous
import jax
import jax.numpy as jnp
from jax import lax
from jax.experimental import pallas as pl
from jax.experimental.pallas import tpu as pltpu
from jax.experimental.pallas import tpu_sc as plsc

VOCAB = 100000
EMB = 64
CLS = 4
BATCH = 4096
HIST = 200

LANES = 16
NWORKERS = 32
ITEMS = BATCH // NWORKERS

VP = 102400
N16 = VP // 16
ICHUNK = 3200
BN = 20480

NSLOTS = 8
NGROUPS = HIST // NSLOTS
ROWPAD = 208
ROWW = 8


def _project_body(w_ref, t_ref, o0, o1, o2, o3):
    o = jax.lax.dot_general(
        w_ref[...],
        t_ref[...],
        (((1,), (0,)), ((), ())),
        preferred_element_type=jnp.float32,
    )
    for c, o_ref in enumerate((o0, o1, o2, o3)):
        o_ref[...] = o[c, :]


def _project(w8, t_t):
    grid = VP // BN
    out = jax.ShapeDtypeStruct((VP,), jnp.float32)
    return pl.pallas_call(
        _project_body,
        grid=(grid,),
        in_specs=[
            pl.BlockSpec((8, EMB), lambda i: (0, 0)),
            pl.BlockSpec((EMB, BN), lambda i: (0, i)),
        ],
        out_specs=[pl.BlockSpec((BN,), lambda i: (i,)) for _ in range(CLS)],
        out_shape=[out, out, out, out],
    )(w8, t_t)


def _sc_body(p0, p1, p2, p3, data_hbm, out_hbm,
             rowidx_v, idx_v, cols_v, ichunk_v, accs, shared, sems, ilsem):
    wid = lax.axis_index("s") * 2 + lax.axis_index("c")
    sid = lax.axis_index("s")
    base = wid * ITEMS

    for k in range(ROWPAD // 16):
        r = jnp.minimum(
            (lax.iota(jnp.int32, 16) + 16 * k) * NWORKERS + wid,
            BATCH * HIST // 128 - 1,
        )
        rowidx_v[pl.ds(16 * k, 16)] = r
    pltpu.async_copy(data_hbm.at[rowidx_v], idx_v, sems[0])

    for h in range(N16 // ICHUNK):
        vb = sid * N16 + h * ICHUNK
        for c, src in enumerate((p0, p1, p2, p3)):
            pltpu.async_copy(src.at[pl.ds(vb, ICHUNK)], cols_v.at[c], ilsem)
        for c in range(CLS):
            pltpu.make_async_copy(p0.at[pl.ds(vb, ICHUNK)], cols_v.at[c], ilsem).wait()
        for c in range(CLS):
            ccol = jnp.full((16,), c, jnp.int32)

            @pl.loop(0, ICHUNK // LANES, unroll=4)
            def _(k):
                rows = lax.iota(jnp.int32, 16) + k * LANES
                vals = cols_v[c, pl.ds(k * LANES, LANES)]
                plsc.store_scatter(ichunk_v, [rows, ccol], vals)

        pltpu.sync_copy(ichunk_v, shared.at[pl.ds(vb, ICHUNK)])

    plsc.subcore_barrier()
    pltpu.make_async_copy(data_hbm.at[rowidx_v], idx_v, sems[0]).wait()

    for g in range(NSLOTS):
        pltpu.async_copy(shared.at[idx_v.at[g]], accs[g], sems[g])

    @pl.loop(1, NGROUPS)
    def _(i):
        for g in range(NSLOTS):
            l = i * NSLOTS + g
            pltpu.make_async_copy(shared.at[idx_v.at[g]], accs[g], sems[g]).wait()
            pltpu.async_copy(shared.at[idx_v.at[l]], accs[g], sems[g], add=True)

    for g in range(NSLOTS):
        pltpu.make_async_copy(shared.at[idx_v.at[g]], accs[g], sems[g]).wait()

    @pl.loop(0, ITEMS * ROWW // 16)
    def _(k):
        q = lax.iota(jnp.int32, 16) + k * 16
        rows = lax.shift_right_logical(q, 3)
        ccs = lax.bitwise_and(q, 7)
        tot = plsc.load_gather(accs[0], [rows, ccs])
        for g in range(1, NSLOTS):
            tot = tot + plsc.load_gather(accs[g], [rows, ccs])
        plsc.store_scatter(accs[0], [rows, ccs], tot)

    pltpu.sync_copy(accs[0], out_hbm.at[pl.ds(base, ITEMS)])


def _pooled_lookup(p0, p1, p2, p3, d128):
    mesh = plsc.VectorSubcoreMesh(core_axis_name="c", subcore_axis_name="s")
    scratch = (
        pltpu.VMEM((ROWPAD,), jnp.int32),
        pltpu.VMEM((ROWPAD, ITEMS), jnp.int32),
        pltpu.VMEM((CLS, ICHUNK), jnp.float32),
        pltpu.VMEM((ICHUNK, ROWW), jnp.float32),
        tuple(pltpu.VMEM((ITEMS, ROWW), jnp.float32) for _ in range(NSLOTS)),
        pltpu.VMEM_SHARED((VP, ROWW), jnp.float32),
        tuple(pltpu.SemaphoreType.DMA for _ in range(NSLOTS)),
        pltpu.SemaphoreType.DMA,
    )
    f = pl.kernel(
        _sc_body,
        out_type=jax.ShapeDtypeStruct((BATCH, ROWW), jnp.float32),
        mesh=mesh,
        scratch_types=scratch,
        compiler_params=pltpu.CompilerParams(
            use_tc_tiling_on_sc=False, needs_layout_passes=False
        ),
    )
    return f(p0, p1, p2, p3, d128)


@jax.jit
def kernel(data, table, W, b):
    w8 = jnp.zeros((8, EMB), jnp.float32).at[:CLS].set(W / HIST)

    p0, p1, p2, p3 = _project(w8, table.T)

    d128 = data.T.reshape(BATCH * HIST // 128, 128)
    out8 = _pooled_lookup(p0, p1, p2, p3, d128)
    return out8[:, :CLS] + b


# speedup vs baseline: 2.2324x; 1.0002x over previous
import jax
import jax.numpy as jnp
from jax import lax
from jax.experimental import pallas as pl
from jax.experimental.pallas import tpu as pltpu
from jax.experimental.pallas import tpu_sc as plsc

VOCAB = 100000
EMB = 64
CLS = 4
BATCH = 4096
HIST = 200

LANES = 16
NWORKERS = 32
ITEMS = BATCH // NWORKERS

VP = 102400
N16 = VP // 16
ICHUNK = 3200
BN = 25600

NSLOTS = 8
NGROUPS = HIST // NSLOTS
ROWPAD = 208
ROWW = 8


def _project_body(w_ref, t_ref, o0, o1, o2, o3):
    o = jax.lax.dot_general(
        w_ref[...],
        t_ref[...],
        (((1,), (0,)), ((), ())),
        preferred_element_type=jnp.float32,
    )
    for c, o_ref in enumerate((o0, o1, o2, o3)):
        o_ref[...] = o[c, :]


def _project(w8, t_t):
    grid = VP // BN
    out = jax.ShapeDtypeStruct((VP,), jnp.float32)
    return pl.pallas_call(
        _project_body,
        grid=(grid,),
        in_specs=[
            pl.BlockSpec((8, EMB), lambda i: (0, 0)),
            pl.BlockSpec((EMB, BN), lambda i: (0, i)),
        ],
        out_specs=[pl.BlockSpec((BN,), lambda i: (i,)) for _ in range(CLS)],
        out_shape=[out, out, out, out],
    )(w8, t_t)


def _sc_body(p0, p1, p2, p3, data_hbm, out_hbm,
             rowidx_v, idx_v, cols_v, ichunk_v, accs, shared, sems, ilsem):
    wid = lax.axis_index("s") * 2 + lax.axis_index("c")
    sid = lax.axis_index("s")
    base = wid * ITEMS

    for k in range(ROWPAD // 16):
        r = jnp.minimum(
            (lax.iota(jnp.int32, 16) + 16 * k) * NWORKERS + wid,
            BATCH * HIST // 128 - 1,
        )
        rowidx_v[pl.ds(16 * k, 16)] = r
    pltpu.async_copy(data_hbm.at[rowidx_v], idx_v, sems[0])

    for h in range(N16 // ICHUNK):
        vb = sid * N16 + h * ICHUNK
        for c, src in enumerate((p0, p1, p2, p3)):
            pltpu.async_copy(src.at[pl.ds(vb, ICHUNK)], cols_v.at[c], ilsem)
        for c in range(CLS):
            pltpu.make_async_copy(p0.at[pl.ds(vb, ICHUNK)], cols_v.at[c], ilsem).wait()
        for c in range(CLS):
            ccol = jnp.full((16,), c, jnp.int32)

            @pl.loop(0, ICHUNK // LANES, unroll=4)
            def _(k):
                rows = lax.iota(jnp.int32, 16) + k * LANES
                vals = cols_v[c, pl.ds(k * LANES, LANES)]
                plsc.store_scatter(ichunk_v, [rows, ccol], vals)

        pltpu.sync_copy(ichunk_v, shared.at[pl.ds(vb, ICHUNK)])

    plsc.subcore_barrier()
    pltpu.make_async_copy(data_hbm.at[rowidx_v], idx_v, sems[0]).wait()

    for g in range(NSLOTS):
        pltpu.async_copy(shared.at[idx_v.at[g]], accs[g], sems[g])

    @pl.loop(1, NGROUPS)
    def _(i):
        for g in range(NSLOTS):
            l = i * NSLOTS + g
            pltpu.make_async_copy(shared.at[idx_v.at[g]], accs[g], sems[g]).wait()
            pltpu.async_copy(shared.at[idx_v.at[l]], accs[g], sems[g], add=True)

    for g in range(NSLOTS):
        pltpu.make_async_copy(shared.at[idx_v.at[g]], accs[g], sems[g]).wait()

    @pl.loop(0, ITEMS * ROWW // 16)
    def _(k):
        q = lax.iota(jnp.int32, 16) + k * 16
        rows = lax.shift_right_logical(q, 3)
        ccs = lax.bitwise_and(q, 7)
        tot = plsc.load_gather(accs[0], [rows, ccs])
        for g in range(1, NSLOTS):
            tot = tot + plsc.load_gather(accs[g], [rows, ccs])
        plsc.store_scatter(accs[0], [rows, ccs], tot)

    pltpu.sync_copy(accs[0], out_hbm.at[pl.ds(base, ITEMS)])


def _pooled_lookup(p0, p1, p2, p3, d128):
    mesh = plsc.VectorSubcoreMesh(core_axis_name="c", subcore_axis_name="s")
    scratch = (
        pltpu.VMEM((ROWPAD,), jnp.int32),
        pltpu.VMEM((ROWPAD, ITEMS), jnp.int32),
        pltpu.VMEM((CLS, ICHUNK), jnp.float32),
        pltpu.VMEM((ICHUNK, ROWW), jnp.float32),
        tuple(pltpu.VMEM((ITEMS, ROWW), jnp.float32) for _ in range(NSLOTS)),
        pltpu.VMEM_SHARED((VP, ROWW), jnp.float32),
        tuple(pltpu.SemaphoreType.DMA for _ in range(NSLOTS)),
        pltpu.SemaphoreType.DMA,
    )
    f = pl.kernel(
        _sc_body,
        out_type=jax.ShapeDtypeStruct((BATCH, ROWW), jnp.float32),
        mesh=mesh,
        scratch_types=scratch,
        compiler_params=pltpu.CompilerParams(
            use_tc_tiling_on_sc=False, needs_layout_passes=False
        ),
    )
    return f(p0, p1, p2, p3, d128)


@jax.jit
def kernel(data, table, W, b):
    w8 = jnp.zeros((8, EMB), jnp.float32).at[:CLS].set(W / HIST)

    p0, p1, p2, p3 = _project(w8, table.T)

    d128 = data.T.reshape(BATCH * HIST // 128, 128)
    out8 = _pooled_lookup(p0, p1, p2, p3, d128)
    return out8[:, :CLS] + b
